# Initial kernel scaffold; baseline (speedup 1.0000x reference)
import jax
import jax.numpy as jnp
from jax.experimental import pallas as pl


def kernel(x, edge_index, edge_attr, gamma1, beta1, W_msg, b_msg, W_edge, b_edge, gamma2, beta2, W_upd, b_upd):
    raise NotImplementedError("write your pallas kernel here")


# speedup vs baseline: 1.9981x; 1.9981x over previous
import functools

import jax
import jax.numpy as jnp
from jax import lax
from jax.experimental import pallas as pl
from jax.experimental.pallas import tpu as pltpu
from jax.experimental.pallas import tpu_sc as plsc

N = 10000
E = 320000
D = 128
H = 128
WIDE = 144
NTILES = 32
EPT = E // NTILES
C = 80
NCHUNK = EPT // C
RPT = N // 16
ZR = 25


def _msg_body(x_ref, g_ref, b_ref, w_ref, bm_ref, o_ref):
    h = x_ref[...] * g_ref[...] + b_ref[...]
    m = jax.nn.gelu(jnp.dot(h, w_ref[...], preferred_element_type=jnp.float32)
                    + bm_ref[...])
    o_ref[:, :H] = m
    o_ref[:, H:] = jnp.ones((x_ref.shape[0], WIDE - H), jnp.float32)


def _node_messages(x, g1, b1, w, bm):
    blk = 1000
    return pl.pallas_call(
        _msg_body,
        grid=(N // blk,),
        in_specs=[
            pl.BlockSpec((blk, D), lambda i: (i, 0)),
            pl.BlockSpec((1, D), lambda i: (0, 0)),
            pl.BlockSpec((1, D), lambda i: (0, 0)),
            pl.BlockSpec((D, H), lambda i: (0, 0)),
            pl.BlockSpec((1, H), lambda i: (0, 0)),
        ],
        out_specs=pl.BlockSpec((blk, WIDE), lambda i: (i, 0)),
        out_shape=jax.ShapeDtypeStruct((N, WIDE), jnp.float32),
    )(x, g1, b1, w, bm)


def _edge_body(a_ref, w_ref, be_ref, o_ref):
    m = jax.nn.gelu(jnp.dot(a_ref[...], w_ref[...],
                            preferred_element_type=jnp.float32) + be_ref[...])
    o_ref[:, :H] = m
    o_ref[:, H:] = jnp.ones((a_ref.shape[0], WIDE - H), jnp.float32)


def _edge_messages(edge_attr, w, be):
    blk = 8000
    de = edge_attr.shape[1]
    return pl.pallas_call(
        _edge_body,
        grid=(E // blk,),
        in_specs=[
            pl.BlockSpec((blk, de), lambda i: (i, 0)),
            pl.BlockSpec((de, H), lambda i: (0, 0)),
            pl.BlockSpec((1, H), lambda i: (0, 0)),
        ],
        out_specs=pl.BlockSpec((blk, WIDE), lambda i: (i, 0)),
        out_shape=jax.ShapeDtypeStruct((E, WIDE), jnp.float32),
    )(edge_attr, w, be)


_SC_MESH = plsc.VectorSubcoreMesh(core_axis_name="c", subcore_axis_name="s")


@functools.partial(
    pl.kernel,
    out_type=jax.ShapeDtypeStruct((2, N, WIDE), jnp.float32),
    mesh=_SC_MESH,
    compiler_params=pltpu.CompilerParams(use_tc_tiling_on_sc=False),
    scratch_types=[
        pltpu.VMEM((C,), jnp.int32),
        pltpu.VMEM((C,), jnp.int32),
        pltpu.VMEM((C, WIDE), jnp.float32),
        pltpu.VMEM((C, WIDE), jnp.float32),
        pltpu.VMEM((ZR, WIDE), jnp.float32),
        pltpu.VMEM_SHARED((N, WIDE), jnp.float32),
        pltpu.SemaphoreType.DMA,
    ],
)
def _sc_aggregate(y_hbm, er_hbm, src_hbm, dst_hbm, out_hbm,
                  src_v, dst_v, rows_v, er_v, z_v, acc, sem):
    cid = lax.axis_index("c")
    sid = lax.axis_index("s")
    wid = cid * 16 + sid
    ebase = pl.multiple_of(wid * EPT, 8)

    zero16 = jnp.zeros((16,), jnp.float32)

    def _zrow(i, carry):
        for h in range(WIDE // 16):
            z_v[i, pl.ds(h * 16, 16)] = zero16
        return carry

    lax.fori_loop(0, ZR, _zrow, 0)

    def _zcopy(j, carry):
        pltpu.sync_copy(z_v, acc.at[pl.ds(sid * RPT + j * ZR, ZR)])
        return carry

    lax.fori_loop(0, RPT // ZR, _zcopy, 0)
    plsc.subcore_barrier()

    def _chunk(j, carry):
        off = pl.multiple_of(ebase + j * C, 8)
        pltpu.sync_copy(src_hbm.at[pl.ds(off, C)], src_v)
        pltpu.sync_copy(dst_hbm.at[pl.ds(off, C)], dst_v)
        pltpu.async_copy(y_hbm.at[src_v], rows_v, sem).wait()
        pltpu.sync_copy(er_hbm.at[pl.ds(off, C)], er_v)

        def _mul(e, c2):
            for h in range(WIDE // 16):
                sl = pl.ds(h * 16, 16)
                rows_v[e, sl] = rows_v[e, sl] * er_v[e, sl]
            return c2

        lax.fori_loop(0, C, _mul, 0)
        pltpu.sync_copy(rows_v, acc.at[dst_v], add=True)
        return carry

    lax.fori_loop(0, NCHUNK, _chunk, 0)
    plsc.subcore_barrier()

    pltpu.sync_copy(acc.at[pl.ds(sid * RPT, RPT)],
                    out_hbm.at[cid, pl.ds(sid * RPT, RPT)])


def _comb_body(acc_ref, x_ref, g2x_ref, b2x_ref, g2a_ref, b2a_ref,
               wx_ref, wa_ref, bu_ref, o_ref):
    s = acc_ref[0] + acc_ref[1]
    num = s[:, :H]
    cnt = s[:, H:H + 1]
    agg = num / jnp.maximum(cnt, 1.0)
    hx = x_ref[...] * g2x_ref[...] + b2x_ref[...]
    ha = agg * g2a_ref[...] + b2a_ref[...]
    o_ref[...] = jax.nn.gelu(
        jnp.dot(hx, wx_ref[...], preferred_element_type=jnp.float32)
        + jnp.dot(ha, wa_ref[...], preferred_element_type=jnp.float32)
        + bu_ref[...])


def _combine(acc, x, g2x, b2x, g2a, b2a, wx, wa, bu):
    blk = 1000
    return pl.pallas_call(
        _comb_body,
        grid=(N // blk,),
        in_specs=[
            pl.BlockSpec((2, blk, WIDE), lambda i: (0, i, 0)),
            pl.BlockSpec((blk, D), lambda i: (i, 0)),
            pl.BlockSpec((1, D), lambda i: (0, 0)),
            pl.BlockSpec((1, D), lambda i: (0, 0)),
            pl.BlockSpec((1, H), lambda i: (0, 0)),
            pl.BlockSpec((1, H), lambda i: (0, 0)),
            pl.BlockSpec((D, H), lambda i: (0, 0)),
            pl.BlockSpec((H, H), lambda i: (0, 0)),
            pl.BlockSpec((1, H), lambda i: (0, 0)),
        ],
        out_specs=pl.BlockSpec((blk, H), lambda i: (i, 0)),
        out_shape=jax.ShapeDtypeStruct((N, H), jnp.float32),
    )(acc, x, g2x, b2x, g2a, b2a, wx, wa, bu)


def kernel(x, edge_index, edge_attr, gamma1, beta1, W_msg, b_msg,
           W_edge, b_edge, gamma2, beta2, W_upd, b_upd):
    dst = edge_index[0]
    src = edge_index[1]

    y_ext = _node_messages(x, gamma1.reshape(1, D), beta1.reshape(1, D),
                           W_msg, b_msg.reshape(1, H))
    er_ext = _edge_messages(edge_attr, W_edge, b_edge.reshape(1, H))
    acc = _sc_aggregate(y_ext, er_ext, src, dst)
    out = _combine(acc, x,
                   gamma2[:D].reshape(1, D), beta2[:D].reshape(1, D),
                   gamma2[D:].reshape(1, H), beta2[D:].reshape(1, H),
                   W_upd[:D], W_upd[D:], b_upd.reshape(1, H))
    return out


# speedup vs baseline: 3.0949x; 1.5489x over previous
import functools

import jax
import jax.numpy as jnp
from jax import lax
from jax.experimental import pallas as pl
from jax.experimental.pallas import tpu as pltpu
from jax.experimental.pallas import tpu_sc as plsc

N = 10000
E = 320000
D = 128
H = 128
NPAD = 10240
NROW = NPAD // H
NTILES = 32
EPT = E // NTILES
C = 80
NCHUNK = EPT // C
RPT = NPAD // 16
ZR = 32


def _msg_body(x_ref, g_ref, b_ref, w_ref, bm_ref, o_ref):
    h = x_ref[...] * g_ref[...] + b_ref[...]
    o_ref[...] = jax.nn.gelu(
        jnp.dot(h, w_ref[...], preferred_element_type=jnp.float32)
        + bm_ref[...])


def _node_messages(x, g1, b1, w, bm):
    blk = 1000
    return pl.pallas_call(
        _msg_body,
        grid=(N // blk,),
        in_specs=[
            pl.BlockSpec((blk, D), lambda i: (i, 0)),
            pl.BlockSpec((1, D), lambda i: (0, 0)),
            pl.BlockSpec((1, D), lambda i: (0, 0)),
            pl.BlockSpec((D, H), lambda i: (0, 0)),
            pl.BlockSpec((1, H), lambda i: (0, 0)),
        ],
        out_specs=pl.BlockSpec((blk, H), lambda i: (i, 0)),
        out_shape=jax.ShapeDtypeStruct((N, H), jnp.float32),
    )(x, g1, b1, w, bm)


def _edge_body(a_ref, w_ref, be_ref, o_ref):
    o_ref[...] = jax.nn.gelu(
        jnp.dot(a_ref[...], w_ref[...], preferred_element_type=jnp.float32)
        + be_ref[...])


def _edge_messages(edge_attr, w, be):
    blk = 8000
    de = edge_attr.shape[1]
    return pl.pallas_call(
        _edge_body,
        grid=(E // blk,),
        in_specs=[
            pl.BlockSpec((blk, de), lambda i: (i, 0)),
            pl.BlockSpec((de, H), lambda i: (0, 0)),
            pl.BlockSpec((1, H), lambda i: (0, 0)),
        ],
        out_specs=pl.BlockSpec((blk, H), lambda i: (i, 0)),
        out_shape=jax.ShapeDtypeStruct((E, H), jnp.float32),
    )(edge_attr, w, be)


_SC_MESH = plsc.VectorSubcoreMesh(core_axis_name="c", subcore_axis_name="s")


@functools.partial(
    pl.kernel,
    out_type=(jax.ShapeDtypeStruct((2, NPAD, H), jnp.float32),
              jax.ShapeDtypeStruct((NTILES, NROW, H), jnp.float32)),
    mesh=_SC_MESH,
    compiler_params=pltpu.CompilerParams(use_tc_tiling_on_sc=False,
                                         needs_layout_passes=False),
    scratch_types=[
        pltpu.VMEM((C,), jnp.int32),
        pltpu.VMEM((C,), jnp.int32),
        pltpu.VMEM((C, H), jnp.float32),
        pltpu.VMEM((C, H), jnp.float32),
        pltpu.VMEM((ZR, H), jnp.float32),
        pltpu.VMEM((NROW, H), jnp.float32),
        pltpu.VMEM_SHARED((NPAD, H), jnp.float32),
        pltpu.SemaphoreType.DMA,
    ],
)
def _sc_aggregate(y_hbm, er_hbm, src_hbm, dst_hbm, acc_hbm, hist_hbm,
                  src_v, dst_v, rows_v, er_v, z_v, hist_v, acc, sem):
    cid = lax.axis_index("c")
    sid = lax.axis_index("s")
    wid = cid * 16 + sid
    ebase = pl.multiple_of(wid * EPT, 8)

    zero16 = jnp.zeros((16,), jnp.float32)
    ones16 = jnp.ones((16,), jnp.float32)

    def _zzero(i, carry):
        for h in range(H // 16):
            z_v[i, pl.ds(h * 16, 16)] = zero16
        return carry

    lax.fori_loop(0, ZR, _zzero, 0)

    def _hzero(i, carry):
        for h in range(H // 16):
            hist_v[i, pl.ds(h * 16, 16)] = zero16
        return carry

    lax.fori_loop(0, NROW, _hzero, 0)

    def _zcopy(j, carry):
        pltpu.sync_copy(z_v, acc.at[pl.ds(sid * RPT + j * ZR, ZR)])
        return carry

    lax.fori_loop(0, RPT // ZR, _zcopy, 0)
    plsc.subcore_barrier()

    def _chunk(j, carry):
        off = pl.multiple_of(ebase + j * C, 8)
        pltpu.sync_copy(src_hbm.at[pl.ds(off, C)], src_v)
        pltpu.sync_copy(dst_hbm.at[pl.ds(off, C)], dst_v)
        pltpu.async_copy(y_hbm.at[src_v], rows_v, sem).wait()
        pltpu.sync_copy(er_hbm.at[pl.ds(off, C)], er_v)

        def _mul(e, c2):
            for h in range(H // 16):
                sl = pl.ds(h * 16, 16)
                rows_v[e, sl] = rows_v[e, sl] * er_v[e, sl]
            return c2

        lax.fori_loop(0, C, _mul, 0)
        pltpu.sync_copy(rows_v, acc.at[dst_v], add=True)

        for i in range(C // 16):
            d16 = dst_v[pl.ds(i * 16, 16)]
            hi = lax.shift_right_logical(d16, 7)
            lo = lax.bitwise_and(d16, 127)
            plsc.addupdate_scatter(hist_v, [hi, lo], ones16)
        return carry

    lax.fori_loop(0, NCHUNK, _chunk, 0)
    plsc.subcore_barrier()

    pltpu.sync_copy(acc.at[pl.ds(sid * RPT, RPT)],
                    acc_hbm.at[cid, pl.ds(sid * RPT, RPT)])
    pltpu.sync_copy(hist_v, hist_hbm.at[wid])


def _comb_body(acc_ref, hist_ref, x_ref, g2x_ref, b2x_ref, g2a_ref, b2a_ref,
               wx_ref, wa_ref, bu_ref, o_ref):
    blk = o_ref.shape[0]
    nrow = blk // H
    num = acc_ref[0] + acc_ref[1]
    cnt = jnp.sum(hist_ref[...], axis=0)
    cnt = jnp.maximum(cnt, 1.0)[:, :, None]
    agg = num.reshape(nrow, H, H) / cnt
    agg = agg.reshape(blk, H)
    hx = x_ref[...] * g2x_ref[...] + b2x_ref[...]
    ha = agg * g2a_ref[...] + b2a_ref[...]
    o_ref[...] = jax.nn.gelu(
        jnp.dot(hx, wx_ref[...], preferred_element_type=jnp.float32)
        + jnp.dot(ha, wa_ref[...], preferred_element_type=jnp.float32)
        + bu_ref[...])


def _combine(acc, hist, x, g2x, b2x, g2a, b2a, wx, wa, bu):
    blk = 1024
    nrow = blk // H
    return pl.pallas_call(
        _comb_body,
        grid=(NPAD // blk,),
        in_specs=[
            pl.BlockSpec((2, blk, H), lambda i: (0, i, 0)),
            pl.BlockSpec((NTILES, nrow, H), lambda i: (0, i, 0)),
            pl.BlockSpec((blk, D), lambda i: (i, 0)),
            pl.BlockSpec((1, D), lambda i: (0, 0)),
            pl.BlockSpec((1, D), lambda i: (0, 0)),
            pl.BlockSpec((1, H), lambda i: (0, 0)),
            pl.BlockSpec((1, H), lambda i: (0, 0)),
            pl.BlockSpec((D, H), lambda i: (0, 0)),
            pl.BlockSpec((H, H), lambda i: (0, 0)),
            pl.BlockSpec((1, H), lambda i: (0, 0)),
        ],
        out_specs=pl.BlockSpec((blk, H), lambda i: (i, 0)),
        out_shape=jax.ShapeDtypeStruct((NPAD, H), jnp.float32),
    )(acc, hist, x, g2x, b2x, g2a, b2a, wx, wa, bu)


def kernel(x, edge_index, edge_attr, gamma1, beta1, W_msg, b_msg,
           W_edge, b_edge, gamma2, beta2, W_upd, b_upd):
    dst = edge_index[0]
    src = edge_index[1]

    y = _node_messages(x, gamma1.reshape(1, D), beta1.reshape(1, D),
                       W_msg, b_msg.reshape(1, H))
    er = _edge_messages(edge_attr, W_edge, b_edge.reshape(1, H))
    acc, hist = _sc_aggregate(y, er, src, dst)
    out = _combine(acc, hist, x,
                   gamma2[:D].reshape(1, D), beta2[:D].reshape(1, D),
                   gamma2[D:].reshape(1, H), beta2[D:].reshape(1, H),
                   W_upd[:D], W_upd[D:], b_upd.reshape(1, H))
    return out[:N]
